# baseline (speedup 1.0000x reference)
import jax
import jax.numpy as jnp
from jax.experimental import pallas as pl


def _add_kernel(x_ref, p_ref, o_ref):
    o_ref[...] = x_ref[...] + p_ref[...]


def kernel(x, pos_table):
    B, T, D = x.shape
    bT = 512
    grid = (T // bT,)
    return pl.pallas_call(
        _add_kernel,
        grid=grid,
        in_specs=[
            pl.BlockSpec((B, bT, D), lambda t: (0, t, 0)),
            pl.BlockSpec((bT, D), lambda t: (t, 0)),
        ],
        out_specs=pl.BlockSpec((B, bT, D), lambda t: (0, t, 0)),
        out_shape=jax.ShapeDtypeStruct(x.shape, x.dtype),
    )(x, pos_table[:T])
